# Initial kernel scaffold; baseline (speedup 1.0000x reference)
import jax
import jax.numpy as jnp
from jax.experimental import pallas as pl


def kernel(vertex, edge, nh_indices, center_weight, nh_weight, edge_weight, bias):
    raise NotImplementedError("write your pallas kernel here")


# speedup vs baseline: 1.3217x; 1.3217x over previous
import functools

import jax
import jax.numpy as jnp
from jax import lax
from jax.experimental import pallas as pl
from jax.experimental.pallas import tpu as pltpu
from jax.experimental.pallas import tpu_sc as plsc

N = 10000
DEG = 16
D_IN = 256
D_OUT = 256

NC = 2
NS = 16
NW = NC * NS
N_PAD = 10240
NODES_W = N_PAD // NW
CHUNK = 8
ROWS = CHUNK * DEG
NCHUNK = NODES_W // CHUNK
NVREG = D_IN // 16


def _sc_body(vert_hbm, idxf_hbm, out_hbm, idx_v, rows_v, out_v, sem):
    wid = lax.axis_index("s") * NC + lax.axis_index("c")
    node0 = wid * NODES_W
    pltpu.sync_copy(idxf_hbm.at[pl.ds(node0 * DEG, NODES_W * DEG)], idx_v)

    def chunk_body(c, carry):
        pltpu.async_copy(
            vert_hbm.at[idx_v.at[pl.ds(c * ROWS, ROWS)]], rows_v, sem
        ).wait()

        def node_body(n, carry2):
            base = n * DEG
            acc = [rows_v[base, pl.ds(v * 16, 16)] for v in range(NVREG)]
            for j in range(1, DEG):
                for v in range(NVREG):
                    acc[v] = acc[v] + rows_v[base + j, pl.ds(v * 16, 16)]
            for v in range(NVREG):
                out_v[n, pl.ds(v * 16, 16)] = acc[v]
            return carry2

        lax.fori_loop(0, CHUNK, node_body, 0, unroll=False)
        pltpu.sync_copy(out_v, out_hbm.at[pl.ds(node0 + c * CHUNK, CHUNK)])
        return carry

    lax.fori_loop(0, NCHUNK, chunk_body, 0, unroll=False)


def _make_sc_gather_sum():
    mesh = plsc.VectorSubcoreMesh(
        core_axis_name="c", subcore_axis_name="s", num_cores=NC, num_subcores=NS
    )
    return pl.kernel(
        _sc_body,
        out_type=jax.ShapeDtypeStruct((N_PAD, D_IN), jnp.float32),
        mesh=mesh,
        scratch_types=[
            pltpu.VMEM((NODES_W * DEG,), jnp.int32),
            pltpu.VMEM((ROWS, D_IN), jnp.float32),
            pltpu.VMEM((CHUNK, D_IN), jnp.float32),
            pltpu.SemaphoreType.DMA,
        ],
        name="sc_gather_sum",
    )


def _tc_body(v_ref, s_ref, e_ref, wc_ref, wn_ref, we_ref, b_ref, o_ref):
    acc = jnp.dot(v_ref[...], wc_ref[...], preferred_element_type=jnp.float32)
    acc = acc + jnp.dot(s_ref[...], wn_ref[...], preferred_element_type=jnp.float32)
    acc = acc + jnp.dot(e_ref[...], we_ref[...], preferred_element_type=jnp.float32)
    o_ref[...] = jnp.maximum(acc + b_ref[...], 0.0)


M_BLK = 1000


def _tc_fused(vertex, vsum, edge2d, wc, wn_s, we32, bias2d):
    grid = (N // M_BLK,)
    return pl.pallas_call(
        _tc_body,
        grid=grid,
        in_specs=[
            pl.BlockSpec((M_BLK, D_IN), lambda i: (i, 0)),
            pl.BlockSpec((M_BLK, D_IN), lambda i: (i, 0)),
            pl.BlockSpec((M_BLK, 2 * DEG), lambda i: (i, 0)),
            pl.BlockSpec((D_IN, D_OUT), lambda i: (0, 0)),
            pl.BlockSpec((D_IN, D_OUT), lambda i: (0, 0)),
            pl.BlockSpec((2 * DEG, D_OUT), lambda i: (0, 0)),
            pl.BlockSpec((1, D_OUT), lambda i: (0, 0)),
        ],
        out_specs=pl.BlockSpec((M_BLK, D_OUT), lambda i: (i, 0)),
        out_shape=jax.ShapeDtypeStruct((N, D_OUT), jnp.float32),
        name="tc_fused_gnn",
    )(vertex, vsum, edge2d, wc, wn_s, we32, bias2d)


def kernel(vertex, edge, nh_indices, center_weight, nh_weight, edge_weight, bias):
    idxf = jnp.pad(nh_indices, ((0, N_PAD - N), (0, 0))).reshape(-1)
    vsum = _make_sc_gather_sum()(vertex, idxf)

    inv = 1.0 / DEG
    edge2d = edge.reshape(N, 2 * DEG)
    we32 = jnp.tile(edge_weight, (DEG, 1)) * inv
    wn_s = nh_weight * inv
    bias2d = bias.reshape(1, D_OUT)
    return _tc_fused(vertex, vsum, edge2d, center_weight, wn_s, we32, bias2d)


# speedup vs baseline: 1.6574x; 1.2539x over previous
import functools

import jax
import jax.numpy as jnp
from jax import lax
from jax.experimental import pallas as pl
from jax.experimental.pallas import tpu as pltpu
from jax.experimental.pallas import tpu_sc as plsc

N = 10000
DEG = 16
D_IN = 256
D_OUT = 256

NC = 2
NS = 16
NW = NC * NS
N_PAD = 10240
NODES_W = N_PAD // NW
CHUNK = 8
ROWS = CHUNK * DEG
NCHUNK = NODES_W // CHUNK
NVREG = D_IN // 16


def _sc_body(vert_hbm, idxf_hbm, out_hbm, idx_v, rows0, rows1, out_v, sem0, sem1):
    wid = lax.axis_index("s") * NC + lax.axis_index("c")
    node0 = wid * NODES_W
    pltpu.sync_copy(idxf_hbm.at[pl.ds(node0 * DEG, NODES_W * DEG)], idx_v)

    def start_gather(c, buf, sem):
        pltpu.async_copy(vert_hbm.at[idx_v.at[pl.ds(c * ROWS, ROWS)]], buf, sem)

    def wait_gather(c, buf, sem):
        pltpu.make_async_copy(
            vert_hbm.at[idx_v.at[pl.ds(c * ROWS, ROWS)]], buf, sem
        ).wait()

    def compute(c, buf):
        def node_body(n, carry2):
            base = n * DEG
            acc = [buf[base, pl.ds(v * 16, 16)] for v in range(NVREG)]
            for j in range(1, DEG):
                for v in range(NVREG):
                    acc[v] = acc[v] + buf[base + j, pl.ds(v * 16, 16)]
            for v in range(NVREG):
                out_v[n, pl.ds(v * 16, 16)] = acc[v]
            return carry2

        lax.fori_loop(0, CHUNK, node_body, 0, unroll=False)
        pltpu.sync_copy(out_v, out_hbm.at[pl.ds(node0 + c * CHUNK, CHUNK)])

    start_gather(0, rows0, sem0)

    def pair_body(i, carry):
        c = 2 * i
        start_gather(c + 1, rows1, sem1)
        wait_gather(c, rows0, sem0)
        compute(c, rows0)

        @pl.when(c + 2 < NCHUNK)
        def _():
            start_gather(c + 2, rows0, sem0)

        wait_gather(c + 1, rows1, sem1)
        compute(c + 1, rows1)
        return carry

    lax.fori_loop(0, NCHUNK // 2, pair_body, 0, unroll=False)


def _make_sc_gather_sum():
    mesh = plsc.VectorSubcoreMesh(
        core_axis_name="c", subcore_axis_name="s", num_cores=NC, num_subcores=NS
    )
    return pl.kernel(
        _sc_body,
        out_type=jax.ShapeDtypeStruct((N_PAD, D_IN), jnp.float32),
        mesh=mesh,
        scratch_types=[
            pltpu.VMEM((NODES_W * DEG,), jnp.int32),
            pltpu.VMEM((ROWS, D_IN), jnp.float32),
            pltpu.VMEM((ROWS, D_IN), jnp.float32),
            pltpu.VMEM((CHUNK, D_IN), jnp.float32),
            pltpu.SemaphoreType.DMA,
            pltpu.SemaphoreType.DMA,
        ],
        name="sc_gather_sum",
    )


def _tc_body(v_ref, s_ref, e_ref, wc_ref, wn_ref, we_ref, b_ref, o_ref):
    acc = jnp.dot(v_ref[...], wc_ref[...], preferred_element_type=jnp.float32)
    acc = acc + jnp.dot(s_ref[...], wn_ref[...], preferred_element_type=jnp.float32)
    acc = acc + jnp.dot(e_ref[...], we_ref[...], preferred_element_type=jnp.float32)
    o_ref[...] = jnp.maximum(acc + b_ref[...], 0.0)


M_BLK = 1000


def _tc_fused(vertex, vsum, edge2d, wc, wn_s, we32, bias2d):
    grid = (N // M_BLK,)
    return pl.pallas_call(
        _tc_body,
        grid=grid,
        in_specs=[
            pl.BlockSpec((M_BLK, D_IN), lambda i: (i, 0)),
            pl.BlockSpec((M_BLK, D_IN), lambda i: (i, 0)),
            pl.BlockSpec((M_BLK, 2 * DEG), lambda i: (i, 0)),
            pl.BlockSpec((D_IN, D_OUT), lambda i: (0, 0)),
            pl.BlockSpec((D_IN, D_OUT), lambda i: (0, 0)),
            pl.BlockSpec((2 * DEG, D_OUT), lambda i: (0, 0)),
            pl.BlockSpec((1, D_OUT), lambda i: (0, 0)),
        ],
        out_specs=pl.BlockSpec((M_BLK, D_OUT), lambda i: (i, 0)),
        out_shape=jax.ShapeDtypeStruct((N, D_OUT), jnp.float32),
        name="tc_fused_gnn",
    )(vertex, vsum, edge2d, wc, wn_s, we32, bias2d)


def kernel(vertex, edge, nh_indices, center_weight, nh_weight, edge_weight, bias):
    idxf = jnp.pad(nh_indices, ((0, N_PAD - N), (0, 0))).reshape(-1)
    vsum = _make_sc_gather_sum()(vertex, idxf)

    inv = 1.0 / DEG
    edge2d = edge.reshape(N, 2 * DEG)
    we32 = jnp.tile(edge_weight, (DEG, 1)) * inv
    wn_s = nh_weight * inv
    bias2d = bias.reshape(1, D_OUT)
    return _tc_fused(vertex, vsum, edge2d, center_weight, wn_s, we32, bias2d)


# speedup vs baseline: 1.6586x; 1.0007x over previous
import functools

import jax
import jax.numpy as jnp
from jax import lax
from jax.experimental import pallas as pl
from jax.experimental.pallas import tpu as pltpu
from jax.experimental.pallas import tpu_sc as plsc

N = 10000
DEG = 16
D_IN = 256
D_OUT = 256

NC = 2
NS = 16
NW = NC * NS
N_PAD = 10240
NODES_W = N_PAD // NW
CHUNK = 8
ROWS = CHUNK * DEG
NCHUNK = NODES_W // CHUNK
NVREG = D_IN // 16


def _sc_body(vert_hbm, idxf_hbm, out_hbm, idx_v, rows0, rows1, out_v, sem0, sem1):
    wid = lax.axis_index("s") * NC + lax.axis_index("c")
    node0 = wid * NODES_W
    pltpu.sync_copy(idxf_hbm.at[pl.ds(wid * NCHUNK, NCHUNK)], idx_v)

    def start_gather(c, buf, sem):
        pltpu.async_copy(vert_hbm.at[idx_v.at[c]], buf, sem)

    def wait_gather(c, buf, sem):
        pltpu.make_async_copy(vert_hbm.at[idx_v.at[c]], buf, sem).wait()

    def compute(c, buf):
        def node_body(n, carry2):
            base = n * DEG
            acc = [buf[base, pl.ds(v * 16, 16)] for v in range(NVREG)]
            for j in range(1, DEG):
                for v in range(NVREG):
                    acc[v] = acc[v] + buf[base + j, pl.ds(v * 16, 16)]
            for v in range(NVREG):
                out_v[n, pl.ds(v * 16, 16)] = acc[v]
            return carry2

        lax.fori_loop(0, CHUNK, node_body, 0, unroll=False)
        pltpu.sync_copy(out_v, out_hbm.at[pl.ds(node0 + c * CHUNK, CHUNK)])

    start_gather(0, rows0, sem0)

    def pair_body(i, carry):
        c = 2 * i
        start_gather(c + 1, rows1, sem1)
        wait_gather(c, rows0, sem0)
        compute(c, rows0)

        @pl.when(c + 2 < NCHUNK)
        def _():
            start_gather(c + 2, rows0, sem0)

        wait_gather(c + 1, rows1, sem1)
        compute(c + 1, rows1)
        return carry

    lax.fori_loop(0, NCHUNK // 2, pair_body, 0, unroll=False)


def _make_sc_gather_sum():
    mesh = plsc.VectorSubcoreMesh(
        core_axis_name="c", subcore_axis_name="s", num_cores=NC, num_subcores=NS
    )
    return pl.kernel(
        _sc_body,
        out_type=jax.ShapeDtypeStruct((N_PAD, D_IN), jnp.float32),
        mesh=mesh,
        scratch_types=[
            pltpu.VMEM((NCHUNK, ROWS), jnp.int32),
            pltpu.VMEM((ROWS, D_IN), jnp.float32),
            pltpu.VMEM((ROWS, D_IN), jnp.float32),
            pltpu.VMEM((CHUNK, D_IN), jnp.float32),
            pltpu.SemaphoreType.DMA,
            pltpu.SemaphoreType.DMA,
        ],
        name="sc_gather_sum",
    )


def _tc_body(v_ref, s_ref, e_ref, wc_ref, wn_ref, we_ref, b_ref, o_ref):
    acc = jnp.dot(v_ref[...], wc_ref[...], preferred_element_type=jnp.float32)
    acc = acc + jnp.dot(s_ref[...], wn_ref[...], preferred_element_type=jnp.float32)
    acc = acc + jnp.dot(e_ref[...], we_ref[...], preferred_element_type=jnp.float32)
    o_ref[...] = jnp.maximum(acc + b_ref[...], 0.0)


M_BLK = 1000


def _tc_fused(vertex, vsum, edge2d, wc, wn_s, we32, bias2d):
    grid = (N // M_BLK,)
    return pl.pallas_call(
        _tc_body,
        grid=grid,
        in_specs=[
            pl.BlockSpec((M_BLK, D_IN), lambda i: (i, 0)),
            pl.BlockSpec((M_BLK, D_IN), lambda i: (i, 0)),
            pl.BlockSpec((M_BLK, 2 * DEG), lambda i: (i, 0)),
            pl.BlockSpec((D_IN, D_OUT), lambda i: (0, 0)),
            pl.BlockSpec((D_IN, D_OUT), lambda i: (0, 0)),
            pl.BlockSpec((2 * DEG, D_OUT), lambda i: (0, 0)),
            pl.BlockSpec((1, D_OUT), lambda i: (0, 0)),
        ],
        out_specs=pl.BlockSpec((M_BLK, D_OUT), lambda i: (i, 0)),
        out_shape=jax.ShapeDtypeStruct((N, D_OUT), jnp.float32),
        name="tc_fused_gnn",
    )(vertex, vsum, edge2d, wc, wn_s, we32, bias2d)


def kernel(vertex, edge, nh_indices, center_weight, nh_weight, edge_weight, bias):
    idxf = jnp.pad(nh_indices, ((0, N_PAD - N), (0, 0))).reshape(NW * NCHUNK, ROWS)
    vsum = _make_sc_gather_sum()(vertex, idxf)

    inv = 1.0 / DEG
    edge2d = edge.reshape(N, 2 * DEG)
    we32 = jnp.tile(edge_weight, (DEG, 1)) * inv
    wn_s = nh_weight * inv
    bias2d = bias.reshape(1, D_OUT)
    return _tc_fused(vertex, vsum, edge2d, center_weight, wn_s, we32, bias2d)


# speedup vs baseline: 1.6604x; 1.0011x over previous
import functools

import jax
import jax.numpy as jnp
from jax import lax
from jax.experimental import pallas as pl
from jax.experimental.pallas import tpu as pltpu
from jax.experimental.pallas import tpu_sc as plsc

N = 10000
DEG = 16
D_IN = 256
D_OUT = 256

NC = 2
NS = 16
NW = NC * NS
N_PAD = 10240
NODES_W = N_PAD // NW
CHUNK = 8
ROWS = CHUNK * DEG
NCHUNK = NODES_W // CHUNK
NVREG = D_IN // 16


def _sc_body(
    vert_hbm, idxf_hbm, out_hbm, i0, i1, rows0, rows1, out_v, si0, si1, sg0, sg1
):
    wid = lax.axis_index("s") * NC + lax.axis_index("c")
    node0 = wid * NODES_W
    row0 = wid * NCHUNK

    def start_idx(c, ibuf, sem):
        pltpu.async_copy(idxf_hbm.at[row0 + c], ibuf, sem)

    def wait_idx(c, ibuf, sem):
        pltpu.make_async_copy(idxf_hbm.at[row0 + c], ibuf, sem).wait()

    def start_gather(ibuf, buf, sem):
        pltpu.async_copy(vert_hbm.at[ibuf], buf, sem)

    def wait_gather(ibuf, buf, sem):
        pltpu.make_async_copy(vert_hbm.at[ibuf], buf, sem).wait()

    def compute(c, buf):
        def node_body(n, carry2):
            base = n * DEG
            acc = [buf[base, pl.ds(v * 16, 16)] for v in range(NVREG)]
            for j in range(1, DEG):
                for v in range(NVREG):
                    acc[v] = acc[v] + buf[base + j, pl.ds(v * 16, 16)]
            for v in range(NVREG):
                out_v[n, pl.ds(v * 16, 16)] = acc[v]
            return carry2

        lax.fori_loop(0, CHUNK, node_body, 0, unroll=False)
        pltpu.sync_copy(out_v, out_hbm.at[pl.ds(node0 + c * CHUNK, CHUNK)])

    start_idx(0, i0, si0)
    start_idx(1, i1, si1)
    wait_idx(0, i0, si0)
    start_gather(i0, rows0, sg0)

    def pair_body(i, carry):
        c = 2 * i
        wait_idx(c + 1, i1, si1)
        start_gather(i1, rows1, sg1)
        wait_gather(i0, rows0, sg0)

        @pl.when(c + 2 < NCHUNK)
        def _():
            start_idx(c + 2, i0, si0)

        compute(c, rows0)

        @pl.when(c + 2 < NCHUNK)
        def _():
            wait_idx(c + 2, i0, si0)
            start_gather(i0, rows0, sg0)

        wait_gather(i1, rows1, sg1)

        @pl.when(c + 3 < NCHUNK)
        def _():
            start_idx(c + 3, i1, si1)

        compute(c + 1, rows1)
        return carry

    lax.fori_loop(0, NCHUNK // 2, pair_body, 0, unroll=False)


def _make_sc_gather_sum():
    mesh = plsc.VectorSubcoreMesh(
        core_axis_name="c", subcore_axis_name="s", num_cores=NC, num_subcores=NS
    )
    return pl.kernel(
        _sc_body,
        out_type=jax.ShapeDtypeStruct((N_PAD, D_IN), jnp.float32),
        mesh=mesh,
        scratch_types=[
            pltpu.VMEM((ROWS,), jnp.int32),
            pltpu.VMEM((ROWS,), jnp.int32),
            pltpu.VMEM((ROWS, D_IN), jnp.float32),
            pltpu.VMEM((ROWS, D_IN), jnp.float32),
            pltpu.VMEM((CHUNK, D_IN), jnp.float32),
            pltpu.SemaphoreType.DMA,
            pltpu.SemaphoreType.DMA,
            pltpu.SemaphoreType.DMA,
            pltpu.SemaphoreType.DMA,
        ],
        name="sc_gather_sum",
    )


def _tc_body(v_ref, s_ref, e_ref, wc_ref, wn_ref, we_ref, b_ref, o_ref):
    acc = jnp.dot(v_ref[...], wc_ref[...], preferred_element_type=jnp.float32)
    acc = acc + jnp.dot(s_ref[...], wn_ref[...], preferred_element_type=jnp.float32)
    acc = acc + jnp.dot(e_ref[...], we_ref[...], preferred_element_type=jnp.float32)
    o_ref[...] = jnp.maximum(acc + b_ref[...], 0.0)


M_BLK = 1000


def _tc_fused(vertex, vsum, edge2d, wc, wn_s, we32, bias2d):
    grid = (N // M_BLK,)
    return pl.pallas_call(
        _tc_body,
        grid=grid,
        in_specs=[
            pl.BlockSpec((M_BLK, D_IN), lambda i: (i, 0)),
            pl.BlockSpec((M_BLK, D_IN), lambda i: (i, 0)),
            pl.BlockSpec((M_BLK, 2 * DEG), lambda i: (i, 0)),
            pl.BlockSpec((D_IN, D_OUT), lambda i: (0, 0)),
            pl.BlockSpec((D_IN, D_OUT), lambda i: (0, 0)),
            pl.BlockSpec((2 * DEG, D_OUT), lambda i: (0, 0)),
            pl.BlockSpec((1, D_OUT), lambda i: (0, 0)),
        ],
        out_specs=pl.BlockSpec((M_BLK, D_OUT), lambda i: (i, 0)),
        out_shape=jax.ShapeDtypeStruct((N, D_OUT), jnp.float32),
        name="tc_fused_gnn",
    )(vertex, vsum, edge2d, wc, wn_s, we32, bias2d)


def kernel(vertex, edge, nh_indices, center_weight, nh_weight, edge_weight, bias):
    idxf = jnp.pad(nh_indices, ((0, N_PAD - N), (0, 0))).reshape(NW * NCHUNK, ROWS)
    vsum = _make_sc_gather_sum()(vertex, idxf)

    inv = 1.0 / DEG
    edge2d = edge.reshape(N, 2 * DEG)
    we32 = jnp.tile(edge_weight, (DEG, 1)) * inv
    wn_s = nh_weight * inv
    bias2d = bias.reshape(1, D_OUT)
    return _tc_fused(vertex, vsum, edge2d, center_weight, wn_s, we32, bias2d)


# speedup vs baseline: 4.1766x; 2.5155x over previous
import functools

import jax
import jax.numpy as jnp
from jax import lax
from jax.experimental import pallas as pl
from jax.experimental.pallas import tpu as pltpu
from jax.experimental.pallas import tpu_sc as plsc

N = 10000
DEG = 16
D_IN = 256
D_OUT = 256
DH = D_IN // 2

NC = 2
NS = 16
N_PAD = 10240
NODES_W = N_PAD // NS
CHUNK = 8
ROWS = CHUNK * DEG
NCHUNK = NODES_W // CHUNK
NVREG = DH // 16


def _sc_body(
    vert2_hbm, idxf_hbm, out_hbm, table, i0, i1, rows0, rows1, out_v, si0, si1,
    sg0, sg1, st
):
    cid = lax.axis_index("c")
    sid = lax.axis_index("s")
    node0 = sid * NODES_W
    row0 = sid * NCHUNK

    @pl.when(sid == 0)
    def _():
        pltpu.async_copy(vert2_hbm.at[cid], table, st).wait()

    plsc.subcore_barrier()

    def start_idx(c, ibuf, sem):
        pltpu.async_copy(idxf_hbm.at[row0 + c], ibuf, sem)

    def wait_idx(c, ibuf, sem):
        pltpu.make_async_copy(idxf_hbm.at[row0 + c], ibuf, sem).wait()

    def start_gather(ibuf, buf, sem):
        pltpu.async_copy(table.at[ibuf], buf, sem)

    def wait_gather(ibuf, buf, sem):
        pltpu.make_async_copy(table.at[ibuf], buf, sem).wait()

    def compute(c, buf):
        def node_body(n, carry2):
            base = n * DEG
            acc = [buf[base, pl.ds(v * 16, 16)] for v in range(NVREG)]
            for j in range(1, DEG):
                for v in range(NVREG):
                    acc[v] = acc[v] + buf[base + j, pl.ds(v * 16, 16)]
            for v in range(NVREG):
                out_v[n, pl.ds(v * 16, 16)] = acc[v]
            return carry2

        lax.fori_loop(0, CHUNK, node_body, 0, unroll=False)
        pltpu.sync_copy(out_v, out_hbm.at[cid].at[pl.ds(node0 + c * CHUNK, CHUNK)])

    start_idx(0, i0, si0)
    start_idx(1, i1, si1)
    wait_idx(0, i0, si0)
    start_gather(i0, rows0, sg0)

    def pair_body(i, carry):
        c = 2 * i
        wait_idx(c + 1, i1, si1)
        start_gather(i1, rows1, sg1)
        wait_gather(i0, rows0, sg0)

        @pl.when(c + 2 < NCHUNK)
        def _():
            start_idx(c + 2, i0, si0)

        compute(c, rows0)

        @pl.when(c + 2 < NCHUNK)
        def _():
            wait_idx(c + 2, i0, si0)
            start_gather(i0, rows0, sg0)

        wait_gather(i1, rows1, sg1)

        @pl.when(c + 3 < NCHUNK)
        def _():
            start_idx(c + 3, i1, si1)

        compute(c + 1, rows1)
        return carry

    lax.fori_loop(0, NCHUNK // 2, pair_body, 0, unroll=False)


def _make_sc_gather_sum():
    mesh = plsc.VectorSubcoreMesh(
        core_axis_name="c", subcore_axis_name="s", num_cores=NC, num_subcores=NS
    )
    return pl.kernel(
        _sc_body,
        out_type=jax.ShapeDtypeStruct((NC, N_PAD, DH), jnp.float32),
        mesh=mesh,
        scratch_types=[
            pltpu.VMEM_SHARED((N, DH), jnp.float32),
            pltpu.VMEM((ROWS,), jnp.int32),
            pltpu.VMEM((ROWS,), jnp.int32),
            pltpu.VMEM((ROWS, DH), jnp.float32),
            pltpu.VMEM((ROWS, DH), jnp.float32),
            pltpu.VMEM((CHUNK, DH), jnp.float32),
            pltpu.SemaphoreType.DMA,
            pltpu.SemaphoreType.DMA,
            pltpu.SemaphoreType.DMA,
            pltpu.SemaphoreType.DMA,
            pltpu.SemaphoreType.DMA,
        ],
        name="sc_gather_sum",
    )


def _tc_body(v_ref, sa_ref, sb_ref, e_ref, wc_ref, wna_ref, wnb_ref, we_ref,
             b_ref, o_ref):
    acc = jnp.dot(v_ref[...], wc_ref[...], preferred_element_type=jnp.float32)
    acc = acc + jnp.dot(sa_ref[0], wna_ref[...], preferred_element_type=jnp.float32)
    acc = acc + jnp.dot(sb_ref[0], wnb_ref[...], preferred_element_type=jnp.float32)
    acc = acc + jnp.dot(e_ref[...], we_ref[...], preferred_element_type=jnp.float32)
    o_ref[...] = jnp.maximum(acc + b_ref[...], 0.0)


M_BLK = 1000


def _tc_fused(vertex, vsum2, edge2d, wc, wna, wnb, we32, bias2d):
    grid = (N // M_BLK,)
    return pl.pallas_call(
        _tc_body,
        grid=grid,
        in_specs=[
            pl.BlockSpec((M_BLK, D_IN), lambda i: (i, 0)),
            pl.BlockSpec((1, M_BLK, DH), lambda i: (0, i, 0)),
            pl.BlockSpec((1, M_BLK, DH), lambda i: (1, i, 0)),
            pl.BlockSpec((M_BLK, 2 * DEG), lambda i: (i, 0)),
            pl.BlockSpec((D_IN, D_OUT), lambda i: (0, 0)),
            pl.BlockSpec((DH, D_OUT), lambda i: (0, 0)),
            pl.BlockSpec((DH, D_OUT), lambda i: (0, 0)),
            pl.BlockSpec((2 * DEG, D_OUT), lambda i: (0, 0)),
            pl.BlockSpec((1, D_OUT), lambda i: (0, 0)),
        ],
        out_specs=pl.BlockSpec((M_BLK, D_OUT), lambda i: (i, 0)),
        out_shape=jax.ShapeDtypeStruct((N, D_OUT), jnp.float32),
        name="tc_fused_gnn",
    )(vertex, vsum2, vsum2, edge2d, wc, wna, wnb, we32, bias2d)


def kernel(vertex, edge, nh_indices, center_weight, nh_weight, edge_weight, bias):
    vert2 = vertex.reshape(N, NC, DH).transpose(1, 0, 2)
    idxf = jnp.pad(nh_indices, ((0, N_PAD - N), (0, 0))).reshape(NS * NCHUNK, ROWS)
    vsum2 = _make_sc_gather_sum()(vert2, idxf)

    inv = 1.0 / DEG
    edge2d = edge.reshape(N, 2 * DEG)
    we32 = jnp.tile(edge_weight, (DEG, 1)) * inv
    wn_s = nh_weight * inv
    bias2d = bias.reshape(1, D_OUT)
    return _tc_fused(vertex, vsum2, edge2d, center_weight, wn_s[:DH], wn_s[DH:],
                     we32, bias2d)


# speedup vs baseline: 5.1484x; 1.2327x over previous
import functools

import jax
import jax.numpy as jnp
from jax import lax
from jax.experimental import pallas as pl
from jax.experimental.pallas import tpu as pltpu
from jax.experimental.pallas import tpu_sc as plsc

N = 10000
DEG = 16
D_IN = 256
D_OUT = 256
DH = D_IN // 2

NC = 2
NS = 16
NODES_W = 640
CHUNK = 8
ROWS = CHUNK * DEG
NCHUNK = NODES_W // CHUNK
NIDX_ROWS = N * DEG // ROWS
NCHUNK_LAST = NIDX_ROWS - (NS - 1) * NCHUNK
NVREG = DH // 16


def _sc_body(
    vert_hbm, idxf_hbm, out_hbm, table, i0, i1, rows0, rows1, out_v, si0, si1,
    sg0, sg1, st
):
    cid = lax.axis_index("c")
    sid = lax.axis_index("s")
    node0 = sid * NODES_W
    row0 = sid * NCHUNK
    nchunk_w = NCHUNK - (NCHUNK - NCHUNK_LAST) * (sid == NS - 1)

    @pl.when(sid == 0)
    def _():
        pltpu.async_copy(vert_hbm.at[:, pl.ds(cid * DH, DH)], table, st).wait()

    plsc.subcore_barrier()

    def start_idx(c, ibuf, sem):
        pltpu.async_copy(idxf_hbm.at[row0 + c], ibuf, sem)

    def wait_idx(c, ibuf, sem):
        pltpu.make_async_copy(idxf_hbm.at[row0 + c], ibuf, sem).wait()

    def start_gather(ibuf, buf, sem):
        pltpu.async_copy(table.at[ibuf], buf, sem)

    def wait_gather(ibuf, buf, sem):
        pltpu.make_async_copy(table.at[ibuf], buf, sem).wait()

    def compute(c, buf):
        def node_body(n, carry2):
            base = n * DEG
            acc = [buf[base, pl.ds(v * 16, 16)] for v in range(NVREG)]
            for j in range(1, DEG):
                for v in range(NVREG):
                    acc[v] = acc[v] + buf[base + j, pl.ds(v * 16, 16)]
            for v in range(NVREG):
                out_v[n, pl.ds(v * 16, 16)] = acc[v]
            return carry2

        lax.fori_loop(0, CHUNK, node_body, 0, unroll=False)
        pltpu.sync_copy(out_v, out_hbm.at[cid].at[pl.ds(node0 + c * CHUNK, CHUNK)])

    start_idx(0, i0, si0)
    start_idx(1, i1, si1)
    wait_idx(0, i0, si0)
    start_gather(i0, rows0, sg0)

    def pair_body(i, carry):
        c = 2 * i
        wait_idx(c + 1, i1, si1)
        start_gather(i1, rows1, sg1)
        wait_gather(i0, rows0, sg0)

        @pl.when(c + 2 < nchunk_w)
        def _():
            start_idx(c + 2, i0, si0)

        compute(c, rows0)

        @pl.when(c + 2 < nchunk_w)
        def _():
            wait_idx(c + 2, i0, si0)
            start_gather(i0, rows0, sg0)

        wait_gather(i1, rows1, sg1)

        @pl.when(c + 3 < nchunk_w)
        def _():
            start_idx(c + 3, i1, si1)

        compute(c + 1, rows1)
        return carry

    lax.fori_loop(0, nchunk_w // 2, pair_body, 0, unroll=False)


def _make_sc_gather_sum():
    mesh = plsc.VectorSubcoreMesh(
        core_axis_name="c", subcore_axis_name="s", num_cores=NC, num_subcores=NS
    )
    return pl.kernel(
        _sc_body,
        out_type=jax.ShapeDtypeStruct((NC, N, DH), jnp.float32),
        mesh=mesh,
        scratch_types=[
            pltpu.VMEM_SHARED((N, DH), jnp.float32),
            pltpu.VMEM((ROWS,), jnp.int32),
            pltpu.VMEM((ROWS,), jnp.int32),
            pltpu.VMEM((ROWS, DH), jnp.float32),
            pltpu.VMEM((ROWS, DH), jnp.float32),
            pltpu.VMEM((CHUNK, DH), jnp.float32),
            pltpu.SemaphoreType.DMA,
            pltpu.SemaphoreType.DMA,
            pltpu.SemaphoreType.DMA,
            pltpu.SemaphoreType.DMA,
            pltpu.SemaphoreType.DMA,
        ],
        name="sc_gather_sum",
    )


def _tc_body(v_ref, sa_ref, sb_ref, e_ref, wc_ref, wna_ref, wnb_ref, we_ref,
             b_ref, o_ref):
    acc = jnp.dot(v_ref[...], wc_ref[...], preferred_element_type=jnp.float32)
    acc = acc + jnp.dot(sa_ref[0], wna_ref[...], preferred_element_type=jnp.float32)
    acc = acc + jnp.dot(sb_ref[0], wnb_ref[...], preferred_element_type=jnp.float32)
    acc = acc + jnp.dot(e_ref[...], we_ref[...], preferred_element_type=jnp.float32)
    o_ref[...] = jnp.maximum(acc + b_ref[...], 0.0)


M_BLK = 1000


def _tc_fused(vertex, vsum2, edge2d, wc, wna, wnb, we32, bias2d):
    grid = (N // M_BLK,)
    return pl.pallas_call(
        _tc_body,
        grid=grid,
        in_specs=[
            pl.BlockSpec((M_BLK, D_IN), lambda i: (i, 0)),
            pl.BlockSpec((1, M_BLK, DH), lambda i: (0, i, 0)),
            pl.BlockSpec((1, M_BLK, DH), lambda i: (1, i, 0)),
            pl.BlockSpec((M_BLK, 2 * DEG), lambda i: (i, 0)),
            pl.BlockSpec((D_IN, D_OUT), lambda i: (0, 0)),
            pl.BlockSpec((DH, D_OUT), lambda i: (0, 0)),
            pl.BlockSpec((DH, D_OUT), lambda i: (0, 0)),
            pl.BlockSpec((2 * DEG, D_OUT), lambda i: (0, 0)),
            pl.BlockSpec((1, D_OUT), lambda i: (0, 0)),
        ],
        out_specs=pl.BlockSpec((M_BLK, D_OUT), lambda i: (i, 0)),
        out_shape=jax.ShapeDtypeStruct((N, D_OUT), jnp.float32),
        name="tc_fused_gnn",
    )(vertex, vsum2, vsum2, edge2d, wc, wna, wnb, we32, bias2d)


def kernel(vertex, edge, nh_indices, center_weight, nh_weight, edge_weight, bias):
    idxf = nh_indices.reshape(NIDX_ROWS, ROWS)
    vsum2 = _make_sc_gather_sum()(vertex, idxf)

    inv = 1.0 / DEG
    edge2d = edge.reshape(N, 2 * DEG)
    we32 = jnp.tile(edge_weight, (DEG, 1)) * inv
    wn_s = nh_weight * inv
    bias2d = bias.reshape(1, D_OUT)
    return _tc_fused(vertex, vsum2, edge2d, center_weight, wn_s[:DH], wn_s[DH:],
                     we32, bias2d)


# speedup vs baseline: 5.2154x; 1.0130x over previous
import functools

import jax
import jax.numpy as jnp
from jax import lax
from jax.experimental import pallas as pl
from jax.experimental.pallas import tpu as pltpu
from jax.experimental.pallas import tpu_sc as plsc

N = 10000
DEG = 16
D_IN = 256
D_OUT = 256
DH = D_IN // 2

NC = 2
NS = 16
NODES_W = 640
CHUNK = 8
ROWS = CHUNK * DEG
NCHUNK = NODES_W // CHUNK
NIDX_ROWS = N * DEG // ROWS
NCHUNK_LAST = NIDX_ROWS - (NS - 1) * NCHUNK
NVREG = DH // 16


def _sc_body(
    vert_hbm, idxf_hbm, out_hbm, table, i0, i1, rows0, rows1, out_v, si0, si1,
    sg0, sg1, st
):
    cid = lax.axis_index("c")
    sid = lax.axis_index("s")
    node0 = sid * NODES_W
    row0 = sid * NCHUNK
    nchunk_w = NCHUNK - (NCHUNK - NCHUNK_LAST) * (sid == NS - 1)

    @pl.when(sid == 0)
    def _():
        pltpu.async_copy(vert_hbm.at[:, pl.ds(cid * DH, DH)], table, st).wait()

    plsc.subcore_barrier()

    def start_idx(c, ibuf, sem):
        pltpu.async_copy(idxf_hbm.at[row0 + c], ibuf, sem)

    def wait_idx(c, ibuf, sem):
        pltpu.make_async_copy(idxf_hbm.at[row0 + c], ibuf, sem).wait()

    def start_gather(ibuf, buf, sem):
        pltpu.async_copy(table.at[ibuf], buf, sem)

    def wait_gather(ibuf, buf, sem):
        pltpu.make_async_copy(table.at[ibuf], buf, sem).wait()

    def compute(c, buf):
        def node_body(n, carry2):
            base = n * DEG
            acc = [buf[base, pl.ds(v * 16, 16)] for v in range(NVREG)]
            for j in range(1, DEG):
                for v in range(NVREG):
                    acc[v] = acc[v] + buf[base + j, pl.ds(v * 16, 16)]
            for v in range(NVREG):
                out_v[n, pl.ds(v * 16, 16)] = acc[v]
            return carry2

        lax.fori_loop(0, CHUNK, node_body, 0, unroll=False)
        pltpu.sync_copy(out_v, out_hbm.at[cid].at[pl.ds(node0 + c * CHUNK, CHUNK)])

    start_idx(0, i0, si0)
    start_idx(1, i1, si1)
    wait_idx(0, i0, si0)
    start_gather(i0, rows0, sg0)

    def pair_body(i, carry):
        c = 2 * i
        wait_idx(c + 1, i1, si1)
        start_gather(i1, rows1, sg1)
        wait_gather(i0, rows0, sg0)

        @pl.when(c + 2 < nchunk_w)
        def _():
            start_idx(c + 2, i0, si0)

        compute(c, rows0)

        @pl.when(c + 2 < nchunk_w)
        def _():
            wait_idx(c + 2, i0, si0)
            start_gather(i0, rows0, sg0)

        wait_gather(i1, rows1, sg1)

        @pl.when(c + 3 < nchunk_w)
        def _():
            start_idx(c + 3, i1, si1)

        compute(c + 1, rows1)
        return carry

    lax.fori_loop(0, nchunk_w // 2, pair_body, 0, unroll=False)


def _make_sc_gather_sum():
    mesh = plsc.VectorSubcoreMesh(
        core_axis_name="c", subcore_axis_name="s", num_cores=NC, num_subcores=NS
    )
    return pl.kernel(
        _sc_body,
        out_type=jax.ShapeDtypeStruct((NC, N, DH), jnp.float32),
        mesh=mesh,
        scratch_types=[
            pltpu.VMEM_SHARED((N, DH), jnp.float32),
            pltpu.VMEM((ROWS,), jnp.int32),
            pltpu.VMEM((ROWS,), jnp.int32),
            pltpu.VMEM((ROWS, DH), jnp.float32),
            pltpu.VMEM((ROWS, DH), jnp.float32),
            pltpu.VMEM((CHUNK, DH), jnp.float32),
            pltpu.SemaphoreType.DMA,
            pltpu.SemaphoreType.DMA,
            pltpu.SemaphoreType.DMA,
            pltpu.SemaphoreType.DMA,
            pltpu.SemaphoreType.DMA,
        ],
        name="sc_gather_sum",
    )


M_BLK = 1000


def _tc_part_body(v_ref, e_ref, wc_ref, we_ref, b_ref, o_ref):
    acc = jnp.dot(v_ref[...], wc_ref[...], preferred_element_type=jnp.float32)
    acc = acc + jnp.dot(e_ref[...], we_ref[...], preferred_element_type=jnp.float32)
    o_ref[...] = acc + b_ref[...]


def _tc_part(vertex, edge2d, wc, we32, bias2d):
    return pl.pallas_call(
        _tc_part_body,
        grid=(N // M_BLK,),
        in_specs=[
            pl.BlockSpec((M_BLK, D_IN), lambda i: (i, 0)),
            pl.BlockSpec((M_BLK, 2 * DEG), lambda i: (i, 0)),
            pl.BlockSpec((D_IN, D_OUT), lambda i: (0, 0)),
            pl.BlockSpec((2 * DEG, D_OUT), lambda i: (0, 0)),
            pl.BlockSpec((1, D_OUT), lambda i: (0, 0)),
        ],
        out_specs=pl.BlockSpec((M_BLK, D_OUT), lambda i: (i, 0)),
        out_shape=jax.ShapeDtypeStruct((N, D_OUT), jnp.float32),
        name="tc_part_gnn",
    )(vertex, edge2d, wc, we32, bias2d)


def _tc_final_body(p_ref, sa_ref, sb_ref, wna_ref, wnb_ref, o_ref):
    acc = p_ref[...]
    acc = acc + jnp.dot(sa_ref[0], wna_ref[...], preferred_element_type=jnp.float32)
    acc = acc + jnp.dot(sb_ref[0], wnb_ref[...], preferred_element_type=jnp.float32)
    o_ref[...] = jnp.maximum(acc, 0.0)


def _tc_final(part, vsum2, wna, wnb):
    return pl.pallas_call(
        _tc_final_body,
        grid=(N // M_BLK,),
        in_specs=[
            pl.BlockSpec((M_BLK, D_OUT), lambda i: (i, 0)),
            pl.BlockSpec((1, M_BLK, DH), lambda i: (0, i, 0)),
            pl.BlockSpec((1, M_BLK, DH), lambda i: (1, i, 0)),
            pl.BlockSpec((DH, D_OUT), lambda i: (0, 0)),
            pl.BlockSpec((DH, D_OUT), lambda i: (0, 0)),
        ],
        out_specs=pl.BlockSpec((M_BLK, D_OUT), lambda i: (i, 0)),
        out_shape=jax.ShapeDtypeStruct((N, D_OUT), jnp.float32),
        name="tc_final_gnn",
    )(part, vsum2, vsum2, wna, wnb)


def kernel(vertex, edge, nh_indices, center_weight, nh_weight, edge_weight, bias):
    idxf = nh_indices.reshape(NIDX_ROWS, ROWS)
    vsum2 = _make_sc_gather_sum()(vertex, idxf)

    inv = 1.0 / DEG
    edge2d = edge.reshape(N, 2 * DEG)
    we32 = jnp.tile(edge_weight, (DEG, 1)) * inv
    wn_s = nh_weight * inv
    bias2d = bias.reshape(1, D_OUT)
    part = _tc_part(vertex, edge2d, center_weight, we32, bias2d)
    return _tc_final(part, vsum2, wn_s[:DH], wn_s[DH:])
